# Initial kernel scaffold; baseline (speedup 1.0000x reference)
import jax
import jax.numpy as jnp
from jax.experimental import pallas as pl


def kernel(x, edge_idx, x_drug, edge_idx_drug, x_cir, edge_idx_cir, params):
    raise NotImplementedError("write your pallas kernel here")


# speedup vs baseline: 7.3030x; 7.3030x over previous
import functools

import jax
import jax.numpy as jnp
from jax import lax
from jax.experimental import pallas as pl
from jax.experimental.pallas import tpu as pltpu
from jax.experimental.pallas import tpu_sc as plsc

F32 = jnp.float32
I32 = jnp.int32
D = 256
DH = 128
CHUNK = 128
BM = 512


def _mesh():
    return plsc.VectorSubcoreMesh(core_axis_name="c", subcore_axis_name="s")


def _zero_rows(ref, nrows):
    z = jnp.zeros((16,), F32)

    def row(r, carry):
        for u in range(8):
            ref[r, pl.ds(u * 16, 16)] = z
        return carry

    lax.fori_loop(0, nrows, row, 0)


def _zero_flat(ref, nvec):
    z = jnp.zeros((16,), F32)

    def it(i, carry):
        off = pl.multiple_of(i * 16, 16)
        ref[pl.ds(off, 16)] = z
        return carry

    lax.fori_loop(0, nvec, it, 0)


def _reduce_partials(s, c, part_v, buf_sh, tmp_v, acc_v, out_hbm, Np):
    Q = Np // 16
    pltpu.sync_copy(part_v, buf_sh.at[s])
    plsc.subcore_barrier()
    q0 = pl.multiple_of(s * Q, 8)
    pltpu.sync_copy(buf_sh.at[0, pl.ds(q0, Q)], acc_v)

    def addone(t, carry):
        pltpu.sync_copy(buf_sh.at[t, pl.ds(q0, Q)], tmp_v)

        def vadd(i, cc):
            off = pl.multiple_of(i * 16, 16)
            acc_v[pl.ds(off, 16)] = (acc_v[pl.ds(off, 16)]
                                     + tmp_v[pl.ds(off, 16)])
            return cc

        lax.fori_loop(0, Q // 16, vadd, 0)
        return carry

    lax.fori_loop(1, 16, addone, 0)
    pltpu.sync_copy(acc_v, out_hbm.at[c, pl.ds(q0, Q)])


@functools.lru_cache(None)
def _deg_kernel(Epad, Np):
    epw = Epad // 32
    ng = epw // 16
    Q = Np // 16

    @functools.partial(
        pl.kernel,
        out_type=jax.ShapeDtypeStruct((2, Np), F32),
        mesh=_mesh(),
        compiler_params=pltpu.CompilerParams(needs_layout_passes=False),
        scratch_types=[
            pltpu.VMEM((epw,), I32),
            pltpu.VMEM((Np,), F32),
            pltpu.VMEM((Q,), F32),
            pltpu.VMEM((Q,), F32),
            pltpu.VMEM_SHARED((16, Np), F32),
        ],
    )
    def k(dst_hbm, out_hbm, dst_v, part_v, tmp_v, acc_v, buf_sh):
        c = lax.axis_index("c")
        s = lax.axis_index("s")
        wid = s * 2 + c
        base = wid * epw
        pltpu.sync_copy(dst_hbm.at[pl.ds(base, epw)], dst_v)
        _zero_flat(part_v, Np // 16)
        ones = jnp.ones((16,), F32)

        def body(i, carry):
            off = pl.multiple_of(i * 16, 16)
            dv = dst_v[pl.ds(off, 16)]
            plsc.addupdate_scatter(part_v, [dv], ones)
            return carry

        lax.fori_loop(0, ng, body, 0)
        _reduce_partials(s, c, part_v, buf_sh, tmp_v, acc_v, out_hbm, Np)

    return k


@functools.lru_cache(None)
def _gat_edge_kernel(Epad, Np):
    epw = Epad // 32
    ng = epw // 16
    Q = Np // 16

    @functools.partial(
        pl.kernel,
        out_type=(
            jax.ShapeDtypeStruct((Epad,), F32),
            jax.ShapeDtypeStruct((2, Np), F32),
        ),
        mesh=_mesh(),
        compiler_params=pltpu.CompilerParams(needs_layout_passes=False),
        scratch_types=[
            pltpu.VMEM((epw,), I32),
            pltpu.VMEM((epw,), I32),
            pltpu.VMEM((epw,), F32),
            pltpu.VMEM((Np,), F32),
            pltpu.VMEM((Np,), F32),
            pltpu.VMEM((Np,), F32),
            pltpu.VMEM((Np,), F32),
            pltpu.VMEM((Q,), F32),
            pltpu.VMEM((Q,), F32),
            pltpu.VMEM_SHARED((16, Np), F32),
        ],
    )
    def k(src_hbm, dst_hbm, alpha_hbm, beta_hbm, cmax_hbm,
          ee_hbm, den_hbm,
          src_v, dst_v, ee_v, alpha_v, beta_v, cmax_v, part_v, tmp_v, acc_v,
          buf_sh):
        c = lax.axis_index("c")
        s = lax.axis_index("s")
        wid = s * 2 + c
        base = wid * epw
        pltpu.sync_copy(src_hbm.at[pl.ds(base, epw)], src_v)
        pltpu.sync_copy(dst_hbm.at[pl.ds(base, epw)], dst_v)
        pltpu.sync_copy(alpha_hbm, alpha_v)
        pltpu.sync_copy(beta_hbm, beta_v)
        pltpu.sync_copy(cmax_hbm, cmax_v)
        _zero_flat(part_v, Np // 16)

        def body(i, carry):
            off = pl.multiple_of(i * 16, 16)
            sv = src_v[pl.ds(off, 16)]
            dv = dst_v[pl.ds(off, 16)]
            a = plsc.load_gather(alpha_v, [sv])
            b = plsc.load_gather(beta_v, [dv])
            cm = plsc.load_gather(cmax_v, [dv])
            e = a + b
            e = jnp.where(e >= 0.0, e, 0.2 * e)
            ee = jnp.exp(e - cm)
            ee_v[pl.ds(off, 16)] = ee
            plsc.addupdate_scatter(part_v, [dv], ee)
            return carry

        lax.fori_loop(0, ng, body, 0)
        pltpu.sync_copy(ee_v, ee_hbm.at[pl.ds(base, epw)])
        _reduce_partials(s, c, part_v, buf_sh, tmp_v, acc_v, den_hbm, Np)

    return k


@functools.lru_cache(None)
def _edge_agg_kernel(Epad, Np, weighted):
    ept = Epad // 16
    nch = ept // CHUNK
    rpt = Np // 16

    scratch = [
        pltpu.VMEM((CHUNK,), I32),
        pltpu.VMEM((CHUNK,), I32),
        pltpu.VMEM((CHUNK, 128), F32),
        pltpu.VMEM((8, 128), F32),
        pltpu.VMEM_SHARED((Np, 128), F32),
    ]
    if weighted:
        scratch.insert(2, pltpu.VMEM((CHUNK,), F32))

    def body(*refs):
        if weighted:
            (tbl0, tbl1, src_hbm, dst_hbm, ew_hbm, out_hbm,
             sidx_v, didx_v, ew_v, rows_v, zbuf, acc_sh) = refs
        else:
            (tbl0, tbl1, src_hbm, dst_hbm, out_hbm,
             sidx_v, didx_v, rows_v, zbuf, acc_sh) = refs
            ew_hbm = ew_v = None
        c = lax.axis_index("c")
        s = lax.axis_index("s")
        _zero_rows(zbuf, 8)

        def zacc(j, carry):
            r0 = pl.multiple_of(s * rpt + j * 8, 8)
            pltpu.sync_copy(zbuf, acc_sh.at[pl.ds(r0, 8)])
            return carry

        lax.fori_loop(0, rpt // 8, zacc, 0)
        plsc.subcore_barrier()

        def run(tbl):
            def chunk(i, carry):
                eb = pl.multiple_of(s * ept + i * CHUNK, CHUNK)
                pltpu.sync_copy(src_hbm.at[pl.ds(eb, CHUNK)], sidx_v)
                pltpu.sync_copy(dst_hbm.at[pl.ds(eb, CHUNK)], didx_v)
                pltpu.sync_copy(tbl.at[sidx_v], rows_v)
                if weighted:
                    pltpu.sync_copy(ew_hbm.at[pl.ds(eb, CHUNK)], ew_v)

                    def rowgrp(g, cc):
                        goff = pl.multiple_of(g * 16, 16)
                        wvec = ew_v[pl.ds(goff, 16)]
                        for j in range(16):
                            w = wvec[j]
                            r = goff + j
                            for u in range(8):
                                rows_v[r, pl.ds(u * 16, 16)] = (
                                    rows_v[r, pl.ds(u * 16, 16)] * w)
                        return cc

                    lax.fori_loop(0, CHUNK // 16, rowgrp, 0)
                pltpu.sync_copy(rows_v, acc_sh.at[didx_v], add=True)
                return carry

            lax.fori_loop(0, nch, chunk, 0)

        @pl.when(c == 0)
        def _():
            run(tbl0)

        @pl.when(c == 1)
        def _():
            run(tbl1)

        plsc.subcore_barrier()

        def wout(j, carry):
            r0 = pl.multiple_of(s * rpt + j * 8, 8)
            pltpu.sync_copy(acc_sh.at[pl.ds(r0, 8)], zbuf)
            pltpu.sync_copy(zbuf, out_hbm.at[c, pl.ds(r0, 8)])
            return carry

        lax.fori_loop(0, rpt // 8, wout, 0)

    return pl.kernel(
        body,
        out_type=jax.ShapeDtypeStruct((2, Np, 128), F32),
        mesh=_mesh(),
        compiler_params=pltpu.CompilerParams(needs_layout_passes=False),
        scratch_types=scratch,
    )


def _grid(N):
    return (N + BM - 1) // BM


@functools.lru_cache(None)
def _sage_post(N):
    def body(agg_ref, d0_ref, d1_ref, x_ref, wn_ref, bn_ref, ws_ref, o_ref):
        deg = d0_ref[...] + d1_ref[...]
        deg = jnp.maximum(deg, 1.0)
        a = agg_ref[...] / deg
        o_ref[...] = (jnp.dot(a, wn_ref[...], preferred_element_type=F32)
                      + bn_ref[...]
                      + jnp.dot(x_ref[...], ws_ref[...],
                                preferred_element_type=F32))

    return pl.pallas_call(
        body,
        grid=(_grid(N),),
        in_specs=[
            pl.BlockSpec((BM, D), lambda i: (i, 0)),
            pl.BlockSpec((BM, 1), lambda i: (i, 0)),
            pl.BlockSpec((BM, 1), lambda i: (i, 0)),
            pl.BlockSpec((BM, D), lambda i: (i, 0)),
            pl.BlockSpec((D, D), lambda i: (0, 0)),
            pl.BlockSpec((1, D), lambda i: (0, 0)),
            pl.BlockSpec((D, D), lambda i: (0, 0)),
        ],
        out_specs=pl.BlockSpec((BM, D), lambda i: (i, 0)),
        out_shape=jax.ShapeDtypeStruct((N, D), F32),
    )


@functools.lru_cache(None)
def _gat_pre(N):
    def body(y_ref, wg_ref, asrc_ref, adst_ref, h_ref, al_ref, be_ref):
        h = jnp.dot(y_ref[...], wg_ref[...], preferred_element_type=F32)
        h_ref[...] = h
        al_ref[...] = jnp.dot(h, asrc_ref[...], preferred_element_type=F32)
        be_ref[...] = jnp.dot(h, adst_ref[...], preferred_element_type=F32)

    return pl.pallas_call(
        body,
        grid=(_grid(N),),
        in_specs=[
            pl.BlockSpec((BM, D), lambda i: (i, 0)),
            pl.BlockSpec((D, D), lambda i: (0, 0)),
            pl.BlockSpec((D, 1), lambda i: (0, 0)),
            pl.BlockSpec((D, 1), lambda i: (0, 0)),
        ],
        out_specs=[
            pl.BlockSpec((BM, D), lambda i: (i, 0)),
            pl.BlockSpec((BM, 1), lambda i: (i, 0)),
            pl.BlockSpec((BM, 1), lambda i: (i, 0)),
        ],
        out_shape=[
            jax.ShapeDtypeStruct((N, D), F32),
            jax.ShapeDtypeStruct((N, 1), F32),
            jax.ShapeDtypeStruct((N, 1), F32),
        ],
    )


@functools.lru_cache(None)
def _gat_scalar(N):
    def body(al_ref, be_ref, cm_ref, el_ref):
        al = al_ref[...]
        be = be_ref[...]
        amax = jnp.max(al)
        t = amax + be
        cm = jnp.where(t >= 0.0, t, 0.2 * t)
        u = al + be
        u = jnp.where(u >= 0.0, u, 0.2 * u)
        cm_ref[...] = cm
        el_ref[...] = jnp.exp(u - cm)

    return pl.pallas_call(
        body,
        grid=(1,),
        in_specs=[
            pl.BlockSpec((N, 1), lambda i: (0, 0)),
            pl.BlockSpec((N, 1), lambda i: (0, 0)),
        ],
        out_specs=[
            pl.BlockSpec((N, 1), lambda i: (0, 0)),
            pl.BlockSpec((N, 1), lambda i: (0, 0)),
        ],
        out_shape=[
            jax.ShapeDtypeStruct((N, 1), F32),
            jax.ShapeDtypeStruct((N, 1), F32),
        ],
    )


@functools.lru_cache(None)
def _gat_post(N):
    def body(nu_ref, d0_ref, d1_ref, el_ref, h_ref, y_ref, wr_ref, bg_ref,
             o_ref):
        el = el_ref[...]
        den = d0_ref[...] + d1_ref[...] + el + 1e-16
        o_ref[...] = ((nu_ref[...] + h_ref[...] * el) / den
                      + bg_ref[...]
                      + jnp.dot(y_ref[...], wr_ref[...],
                                preferred_element_type=F32))

    return pl.pallas_call(
        body,
        grid=(_grid(N),),
        in_specs=[
            pl.BlockSpec((BM, D), lambda i: (i, 0)),
            pl.BlockSpec((BM, 1), lambda i: (i, 0)),
            pl.BlockSpec((BM, 1), lambda i: (i, 0)),
            pl.BlockSpec((BM, 1), lambda i: (i, 0)),
            pl.BlockSpec((BM, D), lambda i: (i, 0)),
            pl.BlockSpec((BM, D), lambda i: (i, 0)),
            pl.BlockSpec((D, D), lambda i: (0, 0)),
            pl.BlockSpec((1, D), lambda i: (0, 0)),
        ],
        out_specs=pl.BlockSpec((BM, D), lambda i: (i, 0)),
        out_shape=jax.ShapeDtypeStruct((N, D), F32),
    )


@functools.lru_cache(None)
def _dec1(M, K):
    def body(a_ref, b_ref, o_ref):
        o_ref[...] = jnp.dot(a_ref[...], b_ref[...],
                             preferred_element_type=F32)

    return pl.pallas_call(
        body,
        grid=(_grid(M),),
        in_specs=[
            pl.BlockSpec((BM, K), lambda i: (i, 0)),
            pl.BlockSpec((K, K), lambda i: (0, 0)),
        ],
        out_specs=pl.BlockSpec((BM, K), lambda i: (i, 0)),
        out_shape=jax.ShapeDtypeStruct((M, K), F32),
    )


@functools.lru_cache(None)
def _dec2(M, Nc, K):
    def body(t_ref, d_ref, o_ref):
        prod = lax.dot_general(t_ref[...], d_ref[...],
                               (((1,), (1,)), ((), ())),
                               preferred_element_type=F32)
        o_ref[...] = jax.nn.sigmoid(prod)

    return pl.pallas_call(
        body,
        grid=(_grid(M), _grid(Nc)),
        in_specs=[
            pl.BlockSpec((BM, K), lambda i, j: (i, 0)),
            pl.BlockSpec((BM, K), lambda i, j: (j, 0)),
        ],
        out_specs=pl.BlockSpec((BM, BM), lambda i, j: (i, j)),
        out_shape=jax.ShapeDtypeStruct((M, Nc), F32),
    )


def _sc_deg(dstp, Epad, Np):
    return _deg_kernel(Epad, Np)(dstp)


def _sc_gat_edge(srcp, dstp, alpha, beta, cmax, Epad, Np):
    return _gat_edge_kernel(Epad, Np)(srcp, dstp, alpha, beta, cmax)


def _sc_edge_agg(tbl0, tbl1, srcp, dstp, ew, Epad, Np):
    if ew is None:
        return _edge_agg_kernel(Epad, Np, False)(tbl0, tbl1, srcp, dstp)
    return _edge_agg_kernel(Epad, Np, True)(tbl0, tbl1, srcp, dstp, ew)


def _pad_split(v, Np):
    n = v.shape[0]
    vp = jnp.concatenate([v, jnp.zeros((Np - n, D), F32)], axis=0)
    return vp[:, :DH], vp[:, DH:]


def _pad1(v, Np):
    v = v.reshape(-1)
    return jnp.concatenate([v, jnp.zeros((Np - v.shape[0],), F32)])


def _branch(x, edge_idx, layers):
    n = x.shape[0]
    e = edge_idx.shape[1]
    Epad = ((e + 2047) // 2048) * 2048
    Np = ((n + 1 + 255) // 256) * 256
    srcp = jnp.concatenate(
        [edge_idx[0].astype(I32), jnp.full((Epad - e,), n, I32)])
    dstp = jnp.concatenate(
        [edge_idx[1].astype(I32), jnp.full((Epad - e,), n, I32)])

    deg2 = _sc_deg(dstp, Epad, Np)
    d0 = deg2[0, :n].reshape(n, 1)
    d1 = deg2[1, :n].reshape(n, 1)

    for p in layers:
        t0, t1 = _pad_split(x, Np)
        acc = _sc_edge_agg(t0, t1, srcp, dstp, None, Epad, Np)
        agg = jnp.concatenate([acc[0, :n], acc[1, :n]], axis=1)
        y = _sage_post(n)(agg, d0, d1, x, p['Wn'], p['bn'].reshape(1, D),
                          p['Ws'])
        h, alpha, beta = _gat_pre(n)(y, p['Wg'], p['asrc'].reshape(D, 1),
                                     p['adst'].reshape(D, 1))
        cmax, eloop = _gat_scalar(n)(alpha, beta)
        ee, den = _sc_gat_edge(srcp, dstp, _pad1(alpha, Np), _pad1(beta, Np),
                               _pad1(cmax, Np), Epad, Np)
        h0, h1 = _pad_split(h, Np)
        accn = _sc_edge_agg(h0, h1, srcp, dstp, ee, Epad, Np)
        numer = jnp.concatenate([accn[0, :n], accn[1, :n]], axis=1)
        x = _gat_post(n)(numer, den[0, :n].reshape(n, 1),
                         den[1, :n].reshape(n, 1), eloop, h, y, p['Wr'],
                         p['bg'].reshape(1, D))
    return x


N_DRUG_ROWS = 2000


def kernel(x, edge_idx, x_drug, edge_idx_drug, x_cir, edge_idx_cir, params):
    eh = _branch(x, edge_idx, params['heter'])
    ed = _branch(x_drug, edge_idx_drug, params['drug'])
    ec = _branch(x_cir, edge_idx_cir, params['cir'])
    Rc = jnp.concatenate([eh[:N_DRUG_ROWS], ed], axis=1)
    Dc = jnp.concatenate([eh[N_DRUG_ROWS:], ec], axis=1)
    K2 = Rc.shape[1]
    T = _dec1(Rc.shape[0], K2)(Rc, params['Wdec'])
    return _dec2(T.shape[0], Dc.shape[0], K2)(T, Dc)
